# Initial kernel scaffold; baseline (speedup 1.0000x reference)
import functools

import jax
import jax.numpy as jnp
from jax import lax
from jax.experimental import pallas as pl
from jax.experimental.pallas import tpu as pltpu
from jax.experimental.pallas import tpu_sc as plsc

N = 10000
E = 320000
D = 128
NC = 2
NS = 16
NW = NC * NS
L = 16

CHUNK = 128
EPW = E // NW
NFULL = EPW // CHUNK
REM = EPW - NFULL * CHUNK

NPAD = 10240
DEG_STRIPE = NPAD // NS
ROWS_PER_TILE = N // NS
ZROWS = 125

_mesh = plsc.VectorSubcoreMesh(
    core_axis_name="c", subcore_axis_name="s", num_cores=NC, num_subcores=NS
)


def _zeros16():
    return jnp.zeros((L,), jnp.float32)


@functools.partial(
    pl.kernel,
    out_type=jax.ShapeDtypeStruct((NC, N), jnp.float32),
    mesh=_mesh,
    scratch_types=[
        pltpu.VMEM_SHARED((NPAD,), jnp.float32),
        pltpu.VMEM((DEG_STRIPE,), jnp.float32),
        pltpu.VMEM((CHUNK,), jnp.float32),
        pltpu.VMEM((REM,), jnp.float32),
        pltpu.VMEM((CHUNK,), jnp.int32),
        pltpu.VMEM((REM,), jnp.int32),
    ],
)
def _deg_kernel(row_hbm, degp_hbm, acc, zbuf, ones_c, ones_r, idx_c, idx_r):
    c = lax.axis_index("c")
    s = lax.axis_index("s")
    wid = c * NS + s

    for j in range(CHUNK // L):
        ones_c[pl.ds(j * L, L)] = jnp.ones((L,), jnp.float32)
    for j in range(REM // L):
        ones_r[pl.ds(j * L, L)] = jnp.ones((L,), jnp.float32)

    def zb(i, carry):
        zbuf[pl.ds(i * L, L)] = _zeros16()
        return carry

    lax.fori_loop(0, DEG_STRIPE // L, zb, 0)
    pltpu.sync_copy(zbuf, acc.at[pl.ds(s * DEG_STRIPE, DEG_STRIPE)])
    plsc.subcore_barrier()

    ebase = wid * EPW

    def body(i, carry):
        base = pl.multiple_of(ebase + i * CHUNK, 8)
        pltpu.sync_copy(row_hbm.at[pl.ds(base, CHUNK)], idx_c)
        pltpu.sync_copy(ones_c, acc.at[idx_c], add=True)
        return carry

    lax.fori_loop(0, NFULL, body, 0)
    if REM:
        base = pl.multiple_of(ebase + NFULL * CHUNK, 8)
        pltpu.sync_copy(row_hbm.at[pl.ds(base, REM)], idx_r)
        pltpu.sync_copy(ones_r, acc.at[idx_r], add=True)

    plsc.subcore_barrier()

    last = N - (NS - 1) * DEG_STRIPE

    @pl.when(s < NS - 1)
    def _():
        pltpu.sync_copy(
            acc.at[pl.ds(s * DEG_STRIPE, DEG_STRIPE)],
            degp_hbm.at[c, pl.ds(s * DEG_STRIPE, DEG_STRIPE)],
        )

    @pl.when(s == NS - 1)
    def _():
        pltpu.sync_copy(
            acc.at[pl.ds((NS - 1) * DEG_STRIPE, last)],
            degp_hbm.at[c, pl.ds((NS - 1) * DEG_STRIPE, last)],
        )


@functools.partial(
    pl.kernel,
    out_type=jax.ShapeDtypeStruct((NC, N, D), jnp.float32),
    mesh=_mesh,
    scratch_types=[
        pltpu.VMEM_SHARED((N, D), jnp.float32),
        pltpu.VMEM((ZROWS, D), jnp.float32),
        pltpu.VMEM((CHUNK, D), jnp.float32),
        pltpu.VMEM((REM, D), jnp.float32),
        pltpu.VMEM((CHUNK,), jnp.int32),
        pltpu.VMEM((CHUNK,), jnp.int32),
        pltpu.VMEM((REM,), jnp.int32),
        pltpu.VMEM((REM,), jnp.int32),
        pltpu.SemaphoreType.DMA,
    ],
)
def _prop_kernel(src_hbm, row_hbm, col_hbm, outp_hbm,
                 acc, zbuf, msg, msg_r, ridx, cidx, ridx_r, cidx_r, gsem):
    c = lax.axis_index("c")
    s = lax.axis_index("s")
    wid = c * NS + s

    def zb(i, carry):
        for j in range(D // L):
            zbuf[i, pl.ds(j * L, L)] = _zeros16()
        return carry

    lax.fori_loop(0, ZROWS, zb, 0)
    for k in range(ROWS_PER_TILE // ZROWS):
        pltpu.sync_copy(zbuf, acc.at[pl.ds(s * ROWS_PER_TILE + k * ZROWS, ZROWS)])
    plsc.subcore_barrier()

    ebase = wid * EPW

    def body(i, carry):
        base = pl.multiple_of(ebase + i * CHUNK, 8)
        pltpu.sync_copy(row_hbm.at[pl.ds(base, CHUNK)], ridx)
        pltpu.sync_copy(col_hbm.at[pl.ds(base, CHUNK)], cidx)
        pltpu.async_copy(src_hbm.at[ridx], msg, gsem).wait()
        pltpu.sync_copy(msg, acc.at[cidx], add=True)
        return carry

    lax.fori_loop(0, NFULL, body, 0)
    if REM:
        base = pl.multiple_of(ebase + NFULL * CHUNK, 8)
        pltpu.sync_copy(row_hbm.at[pl.ds(base, REM)], ridx_r)
        pltpu.sync_copy(col_hbm.at[pl.ds(base, REM)], cidx_r)
        pltpu.async_copy(src_hbm.at[ridx_r], msg_r, gsem).wait()
        pltpu.sync_copy(msg_r, acc.at[cidx_r], add=True)

    plsc.subcore_barrier()
    for k in range(ROWS_PER_TILE // ZROWS):
        r0 = s * ROWS_PER_TILE + k * ZROWS
        pltpu.sync_copy(acc.at[pl.ds(r0, ZROWS)], outp_hbm.at[c, pl.ds(r0, ZROWS)])


R = 1000


def _tc1_body(degA, degB, x, w0, dis_o, ys_o, out0_o):
    deg = degA[...] + degB[...]
    dis = jnp.where(deg > 0.0, lax.rsqrt(deg), 0.0)
    dis_o[...] = dis
    ys_o[...] = x[...] * dis
    out0_o[...] = jnp.dot(x[...], w0[...], preferred_element_type=jnp.float32)


def _tc2_body(pA, pB, dis, out0, w1, out1_o, ys2_o):
    tx1 = -dis[...] * (pA[...] + pB[...])
    out1_o[...] = out0[...] + jnp.dot(tx1, w1[...], preferred_element_type=jnp.float32)
    ys2_o[...] = dis[...] * tx1


def _tc3_body(qA, qB, dis, x, out1, w2, bb, out_o):
    tx2 = -2.0 * dis[...] * (qA[...] + qB[...]) - x[...]
    out_o[...] = (
        out1[...]
        + jnp.dot(tx2, w2[...], preferred_element_type=jnp.float32)
        + bb[...]
    )


def _col_spec():
    return pl.BlockSpec((R, 1), lambda i: (i, 0))


def _mat_spec():
    return pl.BlockSpec((R, D), lambda i: (i, 0))


def _w_spec():
    return pl.BlockSpec((D, D), lambda i: (0, 0))


_tc1 = pl.pallas_call(
    _tc1_body,
    grid=(N // R,),
    in_specs=[_col_spec(), _col_spec(), _mat_spec(), _w_spec()],
    out_specs=[_col_spec(), _mat_spec(), _mat_spec()],
    out_shape=[
        jax.ShapeDtypeStruct((N, 1), jnp.float32),
        jax.ShapeDtypeStruct((N, D), jnp.float32),
        jax.ShapeDtypeStruct((N, D), jnp.float32),
    ],
)

_tc2 = pl.pallas_call(
    _tc2_body,
    grid=(N // R,),
    in_specs=[_mat_spec(), _mat_spec(), _col_spec(), _mat_spec(), _w_spec()],
    out_specs=[_mat_spec(), _mat_spec()],
    out_shape=[
        jax.ShapeDtypeStruct((N, D), jnp.float32),
        jax.ShapeDtypeStruct((N, D), jnp.float32),
    ],
)

_tc3 = pl.pallas_call(
    _tc3_body,
    grid=(N // R,),
    in_specs=[
        _mat_spec(), _mat_spec(), _col_spec(), _mat_spec(), _mat_spec(),
        _w_spec(), pl.BlockSpec((1, D), lambda i: (0, 0)),
    ],
    out_specs=_mat_spec(),
    out_shape=jax.ShapeDtypeStruct((N, D), jnp.float32),
)


@jax.jit
def kernel(x, edge_index, W, b):
    row = edge_index[0]
    col = edge_index[1]

    degp = _deg_kernel(row)
    degA = degp[0].reshape(N, 1)
    degB = degp[1].reshape(N, 1)

    dis, ys, out0 = _tc1(degA, degB, x, W[0])

    p = _prop_kernel(ys, row, col)
    out1, ys2 = _tc2(p[0], p[1], dis, out0, W[1])

    q = _prop_kernel(ys2, row, col)
    out = _tc3(q[0], q[1], dis, x, out1, W[2], b.reshape(1, D))
    return out


# speedup vs baseline: 6.7859x; 6.7859x over previous
import functools

import jax
import jax.numpy as jnp
from jax import lax
from jax.experimental import pallas as pl
from jax.experimental.pallas import tpu as pltpu
from jax.experimental.pallas import tpu_sc as plsc

N = 10000
E = 320000
D = 128
NC = 2
NS = 16
L = 16

CHUNK = 128
TOTCH = E // CHUNK
ROUNDS = TOTCH // NS
EXTRA = TOTCH - ROUNDS * NS

NPAD = 10240
WIN = NPAD // NC
ACCR = WIN + NS
ZSTRIPE = ACCR // NS
ZPAD = 336
ACCR1 = NS * ZPAD
WB = 640

_mesh = plsc.VectorSubcoreMesh(
    core_axis_name="c", subcore_axis_name="s", num_cores=NC, num_subcores=NS
)


def _window_indices(idx_ref, out_ref, lo, trash):
    lov = jnp.broadcast_to(lo, (L,)).astype(jnp.int32)
    tv = jnp.broadcast_to(trash, (L,)).astype(jnp.int32)
    winv = jnp.broadcast_to(jnp.int32(WIN), (L,))
    zero = jnp.zeros((L,), jnp.int32)
    for j in range(CHUNK // L):
        v = idx_ref[pl.ds(j * L, L)] - lov
        inb = (v >= zero) & (v < winv)
        out_ref[pl.ds(j * L, L)] = jnp.where(inb, v, tv)


@functools.partial(
    pl.kernel,
    out_type=jax.ShapeDtypeStruct((NPAD,), jnp.float32),
    mesh=_mesh,
    scratch_types=[
        pltpu.VMEM_SHARED((ACCR1,), jnp.float32),
        pltpu.VMEM((ZPAD,), jnp.float32),
        pltpu.VMEM((CHUNK,), jnp.float32),
        pltpu.VMEM((CHUNK,), jnp.int32),
        pltpu.VMEM((CHUNK,), jnp.int32),
    ],
)
def _deg_kernel(row_hbm, deg_hbm, acc, zbuf, ones_c, idx_c, idx_w):
    c = lax.axis_index("c")
    s = lax.axis_index("s")
    lo = c * WIN
    trash = WIN + s

    for j in range(CHUNK // L):
        ones_c[pl.ds(j * L, L)] = jnp.ones((L,), jnp.float32)

    for j in range(ZPAD // L):
        zbuf[pl.ds(j * L, L)] = jnp.zeros((L,), jnp.float32)
    pltpu.sync_copy(zbuf, acc.at[pl.ds(s * ZPAD, ZPAD)])
    plsc.subcore_barrier()

    def chunk_step(i):
        base = pl.multiple_of((i * NS + s) * CHUNK, 8)
        pltpu.sync_copy(row_hbm.at[pl.ds(base, CHUNK)], idx_c)
        _window_indices(idx_c, idx_w, lo, trash)
        pltpu.sync_copy(ones_c, acc.at[idx_w], add=True)

    def body(i, carry):
        chunk_step(i)
        return carry

    lax.fori_loop(0, ROUNDS, body, 0)

    @pl.when(s < EXTRA)
    def _():
        chunk_step(ROUNDS)

    plsc.subcore_barrier()

    @pl.when(s < WIN // WB)
    def _():
        pltpu.sync_copy(
            acc.at[pl.ds(s * WB, WB)],
            deg_hbm.at[pl.ds(c * WIN + s * WB, WB)],
        )


@functools.partial(
    pl.kernel,
    out_type=jax.ShapeDtypeStruct((NPAD, D), jnp.float32),
    mesh=_mesh,
    scratch_types=[
        pltpu.VMEM_SHARED((ACCR, D), jnp.float32),
        pltpu.VMEM((ZSTRIPE, D), jnp.float32),
        pltpu.VMEM((CHUNK, D), jnp.float32),
        pltpu.VMEM((CHUNK,), jnp.int32),
        pltpu.VMEM((CHUNK,), jnp.int32),
        pltpu.VMEM((CHUNK,), jnp.int32),
        pltpu.SemaphoreType.DMA,
    ],
)
def _prop_kernel(src_hbm, row_hbm, col_hbm, outp_hbm,
                 acc, zbuf, msg, ridx, cidx, cidx_w, gsem):
    c = lax.axis_index("c")
    s = lax.axis_index("s")
    lo = c * WIN
    trash = WIN + s

    def zb(i, carry):
        for j in range(D // L):
            zbuf[i, pl.ds(j * L, L)] = jnp.zeros((L,), jnp.float32)
        return carry

    lax.fori_loop(0, ZSTRIPE, zb, 0)
    pltpu.sync_copy(zbuf, acc.at[pl.ds(s * ZSTRIPE, ZSTRIPE)])
    plsc.subcore_barrier()

    def chunk_step(i):
        base = pl.multiple_of((i * NS + s) * CHUNK, 8)
        pltpu.sync_copy(row_hbm.at[pl.ds(base, CHUNK)], ridx)
        pltpu.sync_copy(col_hbm.at[pl.ds(base, CHUNK)], cidx)
        _window_indices(cidx, cidx_w, lo, trash)
        pltpu.async_copy(src_hbm.at[ridx], msg, gsem).wait()
        pltpu.sync_copy(msg, acc.at[cidx_w], add=True)

    def body(i, carry):
        chunk_step(i)
        return carry

    lax.fori_loop(0, ROUNDS, body, 0)

    @pl.when(s < EXTRA)
    def _():
        chunk_step(ROUNDS)

    plsc.subcore_barrier()

    @pl.when(s < WIN // WB)
    def _():
        pltpu.sync_copy(
            acc.at[pl.ds(s * WB, WB)],
            outp_hbm.at[pl.ds(c * WIN + s * WB, WB)],
        )


R = 1000


def _tc1_body(deg, x, w0, dis_o, out0_o):
    d = deg[...]
    dis_o[...] = jnp.where(d > 0.0, lax.rsqrt(d), 0.0)
    out0_o[...] = jnp.dot(x[...], w0[...], preferred_element_type=jnp.float32)


def _tc2_body(p, dis, out0, w1, out1_o):
    tx1 = -dis[...] * p[...]
    out1_o[...] = out0[...] + jnp.dot(tx1, w1[...], preferred_element_type=jnp.float32)


def _tc3_body(q, dis, x, out1, w2, bb, out_o):
    tx2 = -2.0 * dis[...] * q[...] - x[...]
    out_o[...] = (
        out1[...]
        + jnp.dot(tx2, w2[...], preferred_element_type=jnp.float32)
        + bb[...]
    )


def _col_spec():
    return pl.BlockSpec((R, 1), lambda i: (i, 0))


def _mat_spec():
    return pl.BlockSpec((R, D), lambda i: (i, 0))


def _w_spec():
    return pl.BlockSpec((D, D), lambda i: (0, 0))


_tc1 = pl.pallas_call(
    _tc1_body,
    grid=(N // R,),
    in_specs=[_col_spec(), _mat_spec(), _w_spec()],
    out_specs=[_col_spec(), _mat_spec()],
    out_shape=[
        jax.ShapeDtypeStruct((N, 1), jnp.float32),
        jax.ShapeDtypeStruct((N, D), jnp.float32),
    ],
)

_tc2 = pl.pallas_call(
    _tc2_body,
    grid=(N // R,),
    in_specs=[_mat_spec(), _col_spec(), _mat_spec(), _w_spec()],
    out_specs=_mat_spec(),
    out_shape=jax.ShapeDtypeStruct((N, D), jnp.float32),
)

_tc3 = pl.pallas_call(
    _tc3_body,
    grid=(N // R,),
    in_specs=[
        _mat_spec(), _col_spec(), _mat_spec(), _mat_spec(),
        _w_spec(), pl.BlockSpec((1, D), lambda i: (0, 0)),
    ],
    out_specs=_mat_spec(),
    out_shape=jax.ShapeDtypeStruct((N, D), jnp.float32),
)


@jax.jit
def kernel(x, edge_index, W, b):
    row = edge_index[0]
    col = edge_index[1]

    deg = _deg_kernel(row)[:N].reshape(N, 1)
    dis, out0 = _tc1(deg, x, W[0])

    ys = x * dis
    s1 = _prop_kernel(ys, row, col)[:N]
    out1 = _tc2(s1, dis, out0, W[1])

    ys2 = -(dis * dis) * s1
    s2 = _prop_kernel(ys2, row, col)[:N]
    out = _tc3(s2, dis, x, out1, W[2], b.reshape(1, D))
    return out


# speedup vs baseline: 10.5789x; 1.5590x over previous
import functools

import jax
import jax.numpy as jnp
from jax import lax
from jax.experimental import pallas as pl
from jax.experimental.pallas import tpu as pltpu
from jax.experimental.pallas import tpu_sc as plsc

N = 10000
E = 320000
D = 128
NC = 2
NS = 16
L = 16

CHUNK = 128
G = 4
EPT = E // NS
NBATCH = EPT // (G * CHUNK)
REM = EPT - NBATCH * G * CHUNK

NPAD = 10240
WIN = NPAD // NC
ACCR = WIN + NS
ZSTRIPE = ACCR // NS
ZPAD = 336
ACCR1 = NS * ZPAD
WB = 640

_mesh = plsc.VectorSubcoreMesh(
    core_axis_name="c", subcore_axis_name="s", num_cores=NC, num_subcores=NS
)


def _window_indices(idx_ref, out_ref, lo, trash, offset=0):
    lov = jnp.broadcast_to(lo, (L,)).astype(jnp.int32)
    tv = jnp.broadcast_to(trash, (L,)).astype(jnp.int32)
    winv = jnp.broadcast_to(jnp.int32(WIN), (L,))
    zero = jnp.zeros((L,), jnp.int32)
    for j in range(CHUNK // L):
        v = idx_ref[pl.ds(offset + j * L, L)] - lov
        inb = (v >= zero) & (v < winv)
        out_ref[pl.ds(j * L, L)] = jnp.where(inb, v, tv)


def _window_indices_n(idx_ref, out_ref, lo, trash, n):
    lov = jnp.broadcast_to(lo, (L,)).astype(jnp.int32)
    tv = jnp.broadcast_to(trash, (L,)).astype(jnp.int32)
    winv = jnp.broadcast_to(jnp.int32(WIN), (L,))
    zero = jnp.zeros((L,), jnp.int32)
    for j in range(n // L):
        v = idx_ref[pl.ds(j * L, L)] - lov
        inb = (v >= zero) & (v < winv)
        out_ref[pl.ds(j * L, L)] = jnp.where(inb, v, tv)


@functools.partial(
    pl.kernel,
    out_type=jax.ShapeDtypeStruct((NPAD,), jnp.float32),
    mesh=_mesh,
    scratch_types=[
        pltpu.VMEM_SHARED((ACCR1,), jnp.float32),
        pltpu.VMEM((ZPAD,), jnp.float32),
        pltpu.VMEM((CHUNK,), jnp.float32),
        pltpu.VMEM((G * CHUNK,), jnp.int32),
        pltpu.VMEM((G, CHUNK), jnp.int32),
        pltpu.VMEM((REM,), jnp.int32),
        pltpu.VMEM((REM,), jnp.int32),
        pltpu.SemaphoreType.DMA,
    ],
)
def _deg_kernel(row_hbm, deg_hbm, acc, zbuf, ones_c, rbatch, cw, ridx_r, idx_wr, ssem):
    c = lax.axis_index("c")
    s = lax.axis_index("s")
    lo = c * WIN
    trash = WIN + s
    ebase = s * EPT

    for j in range(CHUNK // L):
        ones_c[pl.ds(j * L, L)] = jnp.ones((L,), jnp.float32)

    for j in range(ZPAD // L):
        zbuf[pl.ds(j * L, L)] = jnp.zeros((L,), jnp.float32)
    pltpu.sync_copy(zbuf, acc.at[pl.ds(s * ZPAD, ZPAD)])
    plsc.subcore_barrier()

    def _wait_scat():
        pltpu.make_async_copy(ones_c, acc.at[cw.at[0]], ssem).wait()

    def batch(i, carry):
        @pl.when(i > 0)
        def _():
            for _k in range(G):
                _wait_scat()

        base = pl.multiple_of(ebase + i * (G * CHUNK), 8)
        pltpu.sync_copy(row_hbm.at[pl.ds(base, G * CHUNK)], rbatch)
        for k in range(G):
            _window_indices(rbatch, cw.at[k], lo, trash, offset=k * CHUNK)
        for k in range(G):
            pltpu.async_copy(ones_c, acc.at[cw.at[k]], ssem, add=True)
        return carry

    lax.fori_loop(0, NBATCH, batch, 0)
    for _k in range(G):
        _wait_scat()

    if REM:
        base = pl.multiple_of(ebase + NBATCH * G * CHUNK, 8)
        pltpu.sync_copy(row_hbm.at[pl.ds(base, REM)], ridx_r)
        _window_indices_n(ridx_r, idx_wr, lo, trash, REM)
        pltpu.sync_copy(ones_c.at[pl.ds(0, REM)], acc.at[idx_wr], add=True)

    plsc.subcore_barrier()

    @pl.when(s < WIN // WB)
    def _():
        pltpu.sync_copy(
            acc.at[pl.ds(s * WB, WB)],
            deg_hbm.at[pl.ds(c * WIN + s * WB, WB)],
        )


@functools.partial(
    pl.kernel,
    out_type=jax.ShapeDtypeStruct((NPAD, D), jnp.float32),
    mesh=_mesh,
    scratch_types=[
        pltpu.VMEM_SHARED((ACCR, D), jnp.float32),
        pltpu.VMEM((ZSTRIPE, D), jnp.float32),
        pltpu.VMEM((2, CHUNK, D), jnp.float32),
        pltpu.VMEM((G * CHUNK,), jnp.int32),
        pltpu.VMEM((G * CHUNK,), jnp.int32),
        pltpu.VMEM((G, CHUNK), jnp.int32),
        pltpu.VMEM((REM, D), jnp.float32),
        pltpu.VMEM((REM,), jnp.int32),
        pltpu.VMEM((REM,), jnp.int32),
        pltpu.VMEM((REM,), jnp.int32),
        pltpu.SemaphoreType.DMA,
        pltpu.SemaphoreType.DMA,
    ],
)
def _prop_kernel(src_hbm, row_hbm, col_hbm, outp_hbm,
                 acc, zbuf, msg, rbatch, cbatch, cw,
                 msg_r, ridx_r, cidx_r, cidx_wr, gsem, ssem):
    c = lax.axis_index("c")
    s = lax.axis_index("s")
    lo = c * WIN
    trash = WIN + s
    ebase = s * EPT

    def zb(i, carry):
        for j in range(D // L):
            zbuf[i, pl.ds(j * L, L)] = jnp.zeros((L,), jnp.float32)
        return carry

    lax.fori_loop(0, ZSTRIPE, zb, 0)
    pltpu.sync_copy(zbuf, acc.at[pl.ds(s * ZSTRIPE, ZSTRIPE)])
    plsc.subcore_barrier()

    def _gslice(k):
        return rbatch.at[pl.ds(k * CHUNK, CHUNK)]

    def _start_gather(k, b):
        return pltpu.async_copy(src_hbm.at[_gslice(k)], msg.at[b], gsem)

    def _wait_gather(b):
        pltpu.make_async_copy(src_hbm.at[_gslice(0)], msg.at[b], gsem).wait()

    def _fire_scatter(k, b):
        pltpu.async_copy(msg.at[b], acc.at[cw.at[k]], ssem, add=True)

    def _wait_scatter():
        pltpu.make_async_copy(msg.at[0], acc.at[cw.at[0]], ssem).wait()

    def batch(i, carry):
        @pl.when(i > 0)
        def _():
            _wait_scatter()
            _wait_scatter()

        base = pl.multiple_of(ebase + i * (G * CHUNK), 8)
        pltpu.sync_copy(row_hbm.at[pl.ds(base, G * CHUNK)], rbatch)
        pltpu.sync_copy(col_hbm.at[pl.ds(base, G * CHUNK)], cbatch)
        for k in range(G):
            _window_indices(cbatch, cw.at[k], lo, trash, offset=k * CHUNK)

        _start_gather(0, 0)
        for k in range(G):
            b = k % 2
            _wait_gather(b)
            if k < G - 1:
                if k >= 1:
                    _wait_scatter()
                _start_gather(k + 1, (k + 1) % 2)
            _fire_scatter(k, b)
        return carry

    lax.fori_loop(0, NBATCH, batch, 0)
    _wait_scatter()
    _wait_scatter()

    if REM:
        base = pl.multiple_of(ebase + NBATCH * G * CHUNK, 8)
        pltpu.sync_copy(row_hbm.at[pl.ds(base, REM)], ridx_r)
        pltpu.sync_copy(col_hbm.at[pl.ds(base, REM)], cidx_r)
        _window_indices_n(cidx_r, cidx_wr, lo, trash, REM)
        pltpu.async_copy(src_hbm.at[ridx_r], msg_r, gsem).wait()
        pltpu.sync_copy(msg_r, acc.at[cidx_wr], add=True)

    plsc.subcore_barrier()

    @pl.when(s < WIN // WB)
    def _():
        pltpu.sync_copy(
            acc.at[pl.ds(s * WB, WB)],
            outp_hbm.at[pl.ds(c * WIN + s * WB, WB)],
        )


R = 1000


def _tc1_body(deg, x, w0, dis_o, out0_o):
    d = deg[...]
    dis_o[...] = jnp.where(d > 0.0, lax.rsqrt(d), 0.0)
    out0_o[...] = jnp.dot(x[...], w0[...], preferred_element_type=jnp.float32)


def _tc2_body(p, dis, out0, w1, out1_o):
    tx1 = -dis[...] * p[...]
    out1_o[...] = out0[...] + jnp.dot(tx1, w1[...], preferred_element_type=jnp.float32)


def _tc3_body(q, dis, x, out1, w2, bb, out_o):
    tx2 = -2.0 * dis[...] * q[...] - x[...]
    out_o[...] = (
        out1[...]
        + jnp.dot(tx2, w2[...], preferred_element_type=jnp.float32)
        + bb[...]
    )


def _col_spec():
    return pl.BlockSpec((R, 1), lambda i: (i, 0))


def _mat_spec():
    return pl.BlockSpec((R, D), lambda i: (i, 0))


def _w_spec():
    return pl.BlockSpec((D, D), lambda i: (0, 0))


_tc1 = pl.pallas_call(
    _tc1_body,
    grid=(N // R,),
    in_specs=[_col_spec(), _mat_spec(), _w_spec()],
    out_specs=[_col_spec(), _mat_spec()],
    out_shape=[
        jax.ShapeDtypeStruct((N, 1), jnp.float32),
        jax.ShapeDtypeStruct((N, D), jnp.float32),
    ],
)

_tc2 = pl.pallas_call(
    _tc2_body,
    grid=(N // R,),
    in_specs=[_mat_spec(), _col_spec(), _mat_spec(), _w_spec()],
    out_specs=_mat_spec(),
    out_shape=jax.ShapeDtypeStruct((N, D), jnp.float32),
)

_tc3 = pl.pallas_call(
    _tc3_body,
    grid=(N // R,),
    in_specs=[
        _mat_spec(), _col_spec(), _mat_spec(), _mat_spec(),
        _w_spec(), pl.BlockSpec((1, D), lambda i: (0, 0)),
    ],
    out_specs=_mat_spec(),
    out_shape=jax.ShapeDtypeStruct((N, D), jnp.float32),
)


@jax.jit
def kernel(x, edge_index, W, b):
    row = edge_index[0]
    col = edge_index[1]

    deg = _deg_kernel(row)[:N].reshape(N, 1)
    dis, out0 = _tc1(deg, x, W[0])

    ys = x * dis
    s1 = _prop_kernel(ys, row, col)[:N]
    out1 = _tc2(s1, dis, out0, W[1])

    ys2 = -(dis * dis) * s1
    s2 = _prop_kernel(ys2, row, col)[:N]
    out = _tc3(s2, dis, x, out1, W[2], b.reshape(1, D))
    return out


# speedup vs baseline: 14.0778x; 1.3307x over previous
import functools

import jax
import jax.numpy as jnp
from jax import lax
from jax.experimental import pallas as pl
from jax.experimental.pallas import tpu as pltpu
from jax.experimental.pallas import tpu_sc as plsc

N = 10000
E = 320000
D = 128
NC = 2
NS = 16
L = 16

CHUNK = 128
G = 4
EPT = E // NS
NBATCH = EPT // (G * CHUNK)
REM = EPT - NBATCH * G * CHUNK

NPAD = 10240
WIN = NPAD // NC
ACCR = WIN + NS
ZSTRIPE = ACCR // NS
ZPAD = 336
ACCR1 = NS * ZPAD
WB = 640

_mesh = plsc.VectorSubcoreMesh(
    core_axis_name="c", subcore_axis_name="s", num_cores=NC, num_subcores=NS
)


def _window_indices(idx_ref, out_ref, lo, trash, offset=0):
    lov = jnp.broadcast_to(lo, (L,)).astype(jnp.int32)
    tv = jnp.broadcast_to(trash, (L,)).astype(jnp.int32)
    winv = jnp.broadcast_to(jnp.int32(WIN), (L,))
    zero = jnp.zeros((L,), jnp.int32)
    for j in range(CHUNK // L):
        v = idx_ref[pl.ds(offset + j * L, L)] - lov
        inb = (v >= zero) & (v < winv)
        out_ref[pl.ds(j * L, L)] = jnp.where(inb, v, tv)


def _window_indices_n(idx_ref, out_ref, lo, trash, n):
    lov = jnp.broadcast_to(lo, (L,)).astype(jnp.int32)
    tv = jnp.broadcast_to(trash, (L,)).astype(jnp.int32)
    winv = jnp.broadcast_to(jnp.int32(WIN), (L,))
    zero = jnp.zeros((L,), jnp.int32)
    for j in range(n // L):
        v = idx_ref[pl.ds(j * L, L)] - lov
        inb = (v >= zero) & (v < winv)
        out_ref[pl.ds(j * L, L)] = jnp.where(inb, v, tv)


@functools.partial(
    pl.kernel,
    out_type=jax.ShapeDtypeStruct((NPAD,), jnp.float32),
    mesh=_mesh,
    scratch_types=[
        pltpu.VMEM_SHARED((ACCR1,), jnp.float32),
        pltpu.VMEM((ZPAD,), jnp.float32),
        pltpu.VMEM((CHUNK,), jnp.float32),
        pltpu.VMEM((G * CHUNK,), jnp.int32),
        pltpu.VMEM((G, CHUNK), jnp.int32),
        pltpu.VMEM((REM,), jnp.int32),
        pltpu.VMEM((REM,), jnp.int32),
        pltpu.SemaphoreType.DMA,
    ],
)
def _deg_kernel(row_hbm, deg_hbm, acc, zbuf, ones_c, rbatch, cw, ridx_r, idx_wr, ssem):
    c = lax.axis_index("c")
    s = lax.axis_index("s")
    lo = c * WIN
    trash = WIN + s
    ebase = s * EPT

    for j in range(CHUNK // L):
        ones_c[pl.ds(j * L, L)] = jnp.ones((L,), jnp.float32)

    for j in range(ZPAD // L):
        zbuf[pl.ds(j * L, L)] = jnp.zeros((L,), jnp.float32)
    pltpu.sync_copy(zbuf, acc.at[pl.ds(s * ZPAD, ZPAD)])
    plsc.subcore_barrier()

    def _wait_scat():
        pltpu.make_async_copy(ones_c, acc.at[cw.at[0]], ssem).wait()

    def batch(i, carry):
        @pl.when(i > 0)
        def _():
            for _k in range(G):
                _wait_scat()

        base = pl.multiple_of(ebase + i * (G * CHUNK), 8)
        pltpu.sync_copy(row_hbm.at[pl.ds(base, G * CHUNK)], rbatch)
        for k in range(G):
            _window_indices(rbatch, cw.at[k], lo, trash, offset=k * CHUNK)
        for k in range(G):
            pltpu.async_copy(ones_c, acc.at[cw.at[k]], ssem, add=True)
        return carry

    lax.fori_loop(0, NBATCH, batch, 0)
    for _k in range(G):
        _wait_scat()

    if REM:
        base = pl.multiple_of(ebase + NBATCH * G * CHUNK, 8)
        pltpu.sync_copy(row_hbm.at[pl.ds(base, REM)], ridx_r)
        _window_indices_n(ridx_r, idx_wr, lo, trash, REM)
        pltpu.sync_copy(ones_c.at[pl.ds(0, REM)], acc.at[idx_wr], add=True)

    plsc.subcore_barrier()

    @pl.when(s < WIN // WB)
    def _():
        pltpu.sync_copy(
            acc.at[pl.ds(s * WB, WB)],
            deg_hbm.at[pl.ds(c * WIN + s * WB, WB)],
        )


@functools.partial(
    pl.kernel,
    out_type=jax.ShapeDtypeStruct((NPAD, D), jnp.float32),
    mesh=_mesh,
    scratch_types=[
        pltpu.VMEM_SHARED((ACCR, D), jnp.float32),
        pltpu.VMEM((CHUNK, D), jnp.float32),
        pltpu.VMEM((4, CHUNK, D), jnp.float32),
        pltpu.VMEM((G * CHUNK,), jnp.int32),
        pltpu.VMEM((G * CHUNK,), jnp.int32),
        pltpu.VMEM((G, CHUNK), jnp.int32),
        pltpu.VMEM((G * CHUNK,), jnp.int32),
        pltpu.VMEM((G * CHUNK,), jnp.int32),
        pltpu.VMEM((G, CHUNK), jnp.int32),
        pltpu.VMEM((REM, D), jnp.float32),
        pltpu.VMEM((REM,), jnp.int32),
        pltpu.VMEM((REM,), jnp.int32),
        pltpu.VMEM((REM,), jnp.int32),
        pltpu.SemaphoreType.DMA,
        pltpu.SemaphoreType.DMA,
    ],
)
def _prop_kernel(src_hbm, row_hbm, col_hbm, outp_hbm,
                 acc, zbuf, msg, rb0, cb0, cw0, rb1, cb1, cw1,
                 msg_r, ridx_r, cidx_r, cidx_wr, gsem, ssem):
    c = lax.axis_index("c")
    s = lax.axis_index("s")
    lo = c * WIN
    trash = WIN + s
    ebase = s * EPT

    def zb(i, carry):
        for j in range(D // L):
            zbuf[i, pl.ds(j * L, L)] = jnp.zeros((L,), jnp.float32)
        return carry

    lax.fori_loop(0, CHUNK, zb, 0)
    pltpu.sync_copy(zbuf, acc.at[pl.ds(s * ZSTRIPE, CHUNK)])
    pltpu.sync_copy(zbuf, acc.at[pl.ds(s * ZSTRIPE + CHUNK, CHUNK)])
    pltpu.sync_copy(
        zbuf.at[pl.ds(0, ZSTRIPE - 2 * CHUNK)],
        acc.at[pl.ds(s * ZSTRIPE + 2 * CHUNK, ZSTRIPE - 2 * CHUNK)],
    )
    plsc.subcore_barrier()

    rbufs = (rb0, rb1)
    cbufs = (cb0, cb1)
    cwbufs = (cw0, cw1)

    def _load_batch(bsel, bidx):
        base = pl.multiple_of(ebase + bidx * (G * CHUNK), 8)
        pltpu.sync_copy(row_hbm.at[pl.ds(base, G * CHUNK)], rbufs[bsel])
        pltpu.sync_copy(col_hbm.at[pl.ds(base, G * CHUNK)], cbufs[bsel])
        for k in range(G):
            _window_indices(cbufs[bsel], cwbufs[bsel].at[k], lo, trash,
                            offset=k * CHUNK)

    def _start_gather(bsel, k, m):
        pltpu.async_copy(
            src_hbm.at[rbufs[bsel].at[pl.ds(k * CHUNK, CHUNK)]], msg.at[m], gsem
        )

    def _wait_gather(m):
        pltpu.make_async_copy(
            src_hbm.at[rb0.at[pl.ds(0, CHUNK)]], msg.at[m], gsem
        ).wait()

    def _fire_scatter(bsel, k, m):
        pltpu.async_copy(msg.at[m], acc.at[cwbufs[bsel].at[k]], ssem, add=True)

    def _wait_scatter():
        pltpu.make_async_copy(msg.at[0], acc.at[cw0.at[0]], ssem).wait()

    _load_batch(0, 0)
    _start_gather(0, 0, 0)
    _start_gather(0, 1, 1)

    def step(i, carry):
        for k in range(8):
            m = k % 4
            _wait_gather(m)
            if k in (0, 1):
                @pl.when(i > 0)
                def _():
                    _wait_scatter()
            else:
                _wait_scatter()
            if k == 1:
                _load_batch(1, 2 * i + 1)
            if k == 5:
                _load_batch(0, 2 * i + 2)
            nb = 0 if k >= 6 else (0 if k < 2 else 1)
            nk = (k + 2) % 4 if k < 2 or k >= 6 else k - 2
            _start_gather(nb, nk, (k + 2) % 4)
            _fire_scatter(1 if k >= 4 else 0, k % 4, m)
        return carry

    lax.fori_loop(0, NBATCH // 2, step, 0)

    for k in range(4):
        m = k % 4
        _wait_gather(m)
        _wait_scatter()
        if k < 2:
            _start_gather(0, k + 2, (k + 2) % 4)
        _fire_scatter(0, k, m)
    _wait_scatter()
    _wait_scatter()

    if REM:
        base = pl.multiple_of(ebase + NBATCH * G * CHUNK, 8)
        pltpu.sync_copy(row_hbm.at[pl.ds(base, REM)], ridx_r)
        pltpu.sync_copy(col_hbm.at[pl.ds(base, REM)], cidx_r)
        _window_indices_n(cidx_r, cidx_wr, lo, trash, REM)
        pltpu.async_copy(src_hbm.at[ridx_r], msg_r, gsem).wait()
        pltpu.sync_copy(msg_r, acc.at[cidx_wr], add=True)

    plsc.subcore_barrier()

    @pl.when(s < WIN // WB)
    def _():
        pltpu.sync_copy(
            acc.at[pl.ds(s * WB, WB)],
            outp_hbm.at[pl.ds(c * WIN + s * WB, WB)],
        )


R = 1000


def _tc1_body(deg, x, w0, dis_o, out0_o):
    d = deg[...]
    dis_o[...] = jnp.where(d > 0.0, lax.rsqrt(d), 0.0)
    out0_o[...] = jnp.dot(x[...], w0[...], preferred_element_type=jnp.float32)


def _tc2_body(p, dis, out0, w1, out1_o):
    tx1 = -dis[...] * p[...]
    out1_o[...] = out0[...] + jnp.dot(tx1, w1[...], preferred_element_type=jnp.float32)


def _tc3_body(q, dis, x, out1, w2, bb, out_o):
    tx2 = -2.0 * dis[...] * q[...] - x[...]
    out_o[...] = (
        out1[...]
        + jnp.dot(tx2, w2[...], preferred_element_type=jnp.float32)
        + bb[...]
    )


def _col_spec():
    return pl.BlockSpec((R, 1), lambda i: (i, 0))


def _mat_spec():
    return pl.BlockSpec((R, D), lambda i: (i, 0))


def _w_spec():
    return pl.BlockSpec((D, D), lambda i: (0, 0))


_tc1 = pl.pallas_call(
    _tc1_body,
    grid=(N // R,),
    in_specs=[_col_spec(), _mat_spec(), _w_spec()],
    out_specs=[_col_spec(), _mat_spec()],
    out_shape=[
        jax.ShapeDtypeStruct((N, 1), jnp.float32),
        jax.ShapeDtypeStruct((N, D), jnp.float32),
    ],
)

_tc2 = pl.pallas_call(
    _tc2_body,
    grid=(N // R,),
    in_specs=[_mat_spec(), _col_spec(), _mat_spec(), _w_spec()],
    out_specs=_mat_spec(),
    out_shape=jax.ShapeDtypeStruct((N, D), jnp.float32),
)

_tc3 = pl.pallas_call(
    _tc3_body,
    grid=(N // R,),
    in_specs=[
        _mat_spec(), _col_spec(), _mat_spec(), _mat_spec(),
        _w_spec(), pl.BlockSpec((1, D), lambda i: (0, 0)),
    ],
    out_specs=_mat_spec(),
    out_shape=jax.ShapeDtypeStruct((N, D), jnp.float32),
)


@jax.jit
def kernel(x, edge_index, W, b):
    row = edge_index[0]
    col = edge_index[1]

    deg = _deg_kernel(row)[:N].reshape(N, 1)
    dis, out0 = _tc1(deg, x, W[0])

    ys = x * dis
    s1 = _prop_kernel(ys, row, col)[:N]
    out1 = _tc2(s1, dis, out0, W[1])

    ys2 = -(dis * dis) * s1
    s2 = _prop_kernel(ys2, row, col)[:N]
    out = _tc3(s2, dis, x, out1, W[2], b.reshape(1, D))
    return out
